# baseline (speedup 1.0000x reference)
import functools

import jax
import jax.numpy as jnp
from jax import lax
from jax.experimental import pallas as pl
from jax.experimental.pallas import tpu as pltpu
from jax.experimental.pallas import tpu_sc as plsc

B, C, H, W = 64, 384, 32, 32
HW = H * W
E = 8
RED = 48
L = 16

BTC = 48
BSC = B - BTC
BB = 8
HB = 2
NCH = H // HB
CV = C // L


def _tc_body(x_ref, w1_ref, b1_ref, w2_ref, b2_ref, wn_ref):
    xs = x_ref[...]
    n = H
    while n > 1:
        n //= 2
        xs = xs[:, :n] + xs[:, n:2 * n]
    ys = xs[:, 0]
    n = W
    while n > 1:
        n //= 2
        ys = ys[:, :n] + ys[:, n:2 * n]
    pooled = ys[:, 0] * (1.0 / HW)
    h = jnp.dot(pooled, w1_ref[...], preferred_element_type=jnp.float32)
    h = h + b1_ref[...]
    h = h * jax.nn.sigmoid(h)
    logits = jnp.dot(h, w2_ref[...], preferred_element_type=jnp.float32)
    logits = logits + b2_ref[...]
    w = jax.nn.softmax(logits, axis=1)
    idx = lax.broadcasted_iota(jnp.int32, (BB, E), 1)
    m1 = jnp.max(w, axis=1, keepdims=True)
    i1 = jnp.min(jnp.where(w == m1, idx, E), axis=1, keepdims=True)
    w_rest = jnp.where(idx == i1, -jnp.inf, w)
    m2 = jnp.max(w_rest, axis=1, keepdims=True)
    i2 = jnp.min(jnp.where(w_rest == m2, idx, E), axis=1, keepdims=True)
    mask = (idx == i1) | (idx == i2)
    wsel = jnp.where(mask, w, 0.0)
    wn_ref[...] = wsel / (jnp.sum(wsel, axis=1, keepdims=True) + 1e-8)


def _sc_body(x_hbm, w1_hbm, b1_hbm, w2p_hbm, b2p_hbm, wn_hbm,
             xb0, xb1, pbuf, w1v, b1v, w2v, b2v, ob, csem0, csem1):
    wid = lax.axis_index("s") * 2 + lax.axis_index("c")

    pltpu.sync_copy(w1_hbm, w1v)
    pltpu.sync_copy(b1_hbm, b1v)
    pltpu.sync_copy(w2p_hbm, w2v)
    pltpu.sync_copy(b2p_hbm, b2v)

    zero = jnp.zeros((L,), jnp.float32)

    @pl.when(wid < BSC)
    def _():
        b = BTC + wid

        def start(ci, buf, sem):
            pltpu.make_async_copy(
                x_hbm.at[b, pl.ds(ci * HB, HB)], buf, sem).start()

        def wait(ci, buf, sem):
            pltpu.make_async_copy(
                x_hbm.at[b, pl.ds(ci * HB, HB)], buf, sem).wait()

        def accumulate(buf, carry):
            def pos(wq, accs):
                accs = list(accs)
                for hh in range(HB):
                    for wu in range(8):
                        ww = wq * 8 + wu
                        for cv in range(CV):
                            accs[cv] = accs[cv] + buf[hh, ww,
                                                      pl.ds(cv * L, L)]
                return tuple(accs)
            return lax.fori_loop(0, W // 8, pos, carry, unroll=False)

        start(0, xb0, csem0)
        accs = (zero,) * CV

        def chunk_pair(g2, carry, ):
            ci = 2 * g2

            @pl.when(ci + 1 < NCH)
            def _():
                start(ci + 1, xb1, csem1)

            wait(ci, xb0, csem0)
            carry = accumulate(xb0, carry)

            @pl.when(ci + 2 < NCH)
            def _():
                start(ci + 2, xb0, csem0)

            wait(ci + 1, xb1, csem1)
            return accumulate(xb1, carry)

        accs = lax.fori_loop(0, NCH // 2, chunk_pair, accs, unroll=False)
        for cv in range(CV):
            pbuf[pl.ds(cv * L, L)] = accs[cv]

        inv = 1.0 / HW

        def h_acc(q, carry):
            hvs = list(carry)
            pv = pbuf[pl.ds(q * L, L)]
            for k in range(L):
                s = pv[k] * inv
                for j in range(RED // L):
                    hvs[j] = hvs[j] + s * w1v[q * L + k, pl.ds(j * L, L)]
            return tuple(hvs)

        hvs = lax.fori_loop(0, C // L, h_acc, (zero,) * (RED // L),
                            unroll=False)
        hjs = []
        for j in range(RED // L):
            hj = hvs[j] + b1v[pl.ds(j * L, L)]
            hj = hj / (1.0 + jnp.exp(-hj))
            hjs.append(hj)

        logits = b2v[...]
        for j in range(RED // L):
            for k in range(L):
                logits = logits + hjs[j][k] * w2v[j * L + k]

        ex = jnp.exp(logits)
        es = [ex[k] for k in range(E)]
        tot = es[0]
        for k in range(1, E):
            tot = tot + es[k]
        rtot = (jnp.ones((L,), jnp.float32) / jnp.full((L,), tot))[0]
        ws = [e * rtot for e in es]
        m1 = ws[0]
        for k in range(1, E):
            m1 = jnp.maximum(m1, ws[k])
        i1 = jnp.int32(E - 1)
        for k in reversed(range(E)):
            i1 = jnp.where(ws[k] == m1, jnp.int32(k), i1)
        wr = [jnp.where(i1 == k, -1.0, ws[k]) for k in range(E)]
        m2 = wr[0]
        for k in range(1, E):
            m2 = jnp.maximum(m2, wr[k])
        i2 = jnp.int32(E - 1)
        for k in reversed(range(E)):
            i2 = jnp.where(wr[k] == m2, jnp.int32(k), i2)
        keep = [(i1 == k) | (i2 == k) for k in range(E)]
        wm = [jnp.where(keep[k], ws[k], 0.0) for k in range(E)]
        den = wm[0]
        for k in range(1, E):
            den = den + wm[k]
        rden = (jnp.ones((L,), jnp.float32)
                / jnp.full((L,), den + 1e-8))[0]
        lanes = lax.iota(jnp.int32, L)
        wnv = zero
        for k in range(E):
            wnv = jnp.where(lanes == k, wm[k] * rden, wnv)
        ob[...] = wnv
        pltpu.sync_copy(ob, wn_hbm.at[wid])


def _bc_body(wn_tc_ref, wn_sc_ref, out_ref):
    wn_tc = wn_tc_ref[...]
    wn_sc = wn_sc_ref[...][:, :E]
    wn = jnp.concatenate([wn_tc, wn_sc], axis=0)
    wnt = wn.T
    out_ref[...] = jnp.broadcast_to(wnt[:, None, None, :], (E, H, W, B))


@jax.jit
def kernel(x, W1, b1, W2, b2):
    xt = jnp.transpose(x, (0, 2, 3, 1))
    w2p = jnp.pad(W2, ((0, 0), (0, L - E)))
    b2p = jnp.concatenate([b2, jnp.full((L - E,), -1e30, jnp.float32)])

    wn_tc = pl.pallas_call(
        _tc_body,
        grid=(BTC // BB,),
        in_specs=[
            pl.BlockSpec((BB, H, W, C), lambda i: (i, 0, 0, 0)),
            pl.BlockSpec((C, RED), lambda i: (0, 0)),
            pl.BlockSpec((1, RED), lambda i: (0, 0)),
            pl.BlockSpec((RED, E), lambda i: (0, 0)),
            pl.BlockSpec((1, E), lambda i: (0, 0)),
        ],
        out_specs=pl.BlockSpec((BB, E), lambda i: (i, 0)),
        out_shape=jax.ShapeDtypeStruct((BTC, E), jnp.float32),
    )(xt, W1, b1.reshape(1, RED), W2, b2.reshape(1, E))

    mesh = plsc.VectorSubcoreMesh(core_axis_name="c", subcore_axis_name="s")
    wn_sc = functools.partial(
        pl.kernel,
        out_type=jax.ShapeDtypeStruct((BSC, L), jnp.float32),
        mesh=mesh,
        compiler_params=pltpu.CompilerParams(
            needs_layout_passes=False, use_tc_tiling_on_sc=True),
        scratch_types=[
            pltpu.VMEM((HB, W, C), jnp.float32),
            pltpu.VMEM((HB, W, C), jnp.float32),
            pltpu.VMEM((C,), jnp.float32),
            pltpu.VMEM((C, RED), jnp.float32),
            pltpu.VMEM((RED,), jnp.float32),
            pltpu.VMEM((RED, L), jnp.float32),
            pltpu.VMEM((L,), jnp.float32),
            pltpu.VMEM((L,), jnp.float32),
            pltpu.SemaphoreType.DMA,
            pltpu.SemaphoreType.DMA,
        ],
    )(_sc_body)(xt, W1, b1, w2p, b2p)

    pout = pl.pallas_call(
        _bc_body,
        in_specs=[
            pl.BlockSpec((BTC, E), lambda: (0, 0)),
            pl.BlockSpec((BSC, L), lambda: (0, 0)),
        ],
        out_specs=pl.BlockSpec((E, H, W, B), lambda: (0, 0, 0, 0)),
        out_shape=jax.ShapeDtypeStruct((E, H, W, B), jnp.float32),
    )(wn_tc, wn_sc)
    return jnp.transpose(pout, (3, 0, 1, 2))


# speedup vs baseline: 1.2423x; 1.2423x over previous
import functools

import jax
import jax.numpy as jnp
from jax import lax
from jax.experimental import pallas as pl
from jax.experimental.pallas import tpu as pltpu
from jax.experimental.pallas import tpu_sc as plsc

B, C, H, W = 64, 384, 32, 32
HW = H * W
E = 8
RED = 48
L = 16

BTC = 56
BSC = B - BTC
BB = 8
HB = 2
NCH = H // HB
CV = C // L


def _tc_body(x_ref, w1_ref, b1_ref, w2_ref, b2_ref, wn_ref):
    xs = x_ref[...]
    n = H
    while n > 1:
        n //= 2
        xs = xs[:, :n] + xs[:, n:2 * n]
    ys = xs[:, 0]
    n = W
    while n > 1:
        n //= 2
        ys = ys[:, :n] + ys[:, n:2 * n]
    pooled = ys[:, 0] * (1.0 / HW)
    h = jnp.dot(pooled, w1_ref[...], preferred_element_type=jnp.float32)
    h = h + b1_ref[...]
    h = h * jax.nn.sigmoid(h)
    logits = jnp.dot(h, w2_ref[...], preferred_element_type=jnp.float32)
    logits = logits + b2_ref[...]
    w = jax.nn.softmax(logits, axis=1)
    idx = lax.broadcasted_iota(jnp.int32, (BB, E), 1)
    m1 = jnp.max(w, axis=1, keepdims=True)
    i1 = jnp.min(jnp.where(w == m1, idx, E), axis=1, keepdims=True)
    w_rest = jnp.where(idx == i1, -jnp.inf, w)
    m2 = jnp.max(w_rest, axis=1, keepdims=True)
    i2 = jnp.min(jnp.where(w_rest == m2, idx, E), axis=1, keepdims=True)
    mask = (idx == i1) | (idx == i2)
    wsel = jnp.where(mask, w, 0.0)
    wn_ref[...] = wsel / (jnp.sum(wsel, axis=1, keepdims=True) + 1e-8)


def _sc_body(x_hbm, w1_hbm, b1_hbm, w2p_hbm, b2p_hbm, wn_hbm,
             xb0, xb1, pbuf, w1v, b1v, w2v, b2v, ob, csem0, csem1):
    wid = lax.axis_index("s") * 2 + lax.axis_index("c")

    pltpu.sync_copy(w1_hbm, w1v)
    pltpu.sync_copy(b1_hbm, b1v)
    pltpu.sync_copy(w2p_hbm, w2v)
    pltpu.sync_copy(b2p_hbm, b2v)

    zero = jnp.zeros((L,), jnp.float32)

    @pl.when(wid < BSC)
    def _():
        b = BTC + wid

        def start(ci, buf, sem):
            pltpu.make_async_copy(
                x_hbm.at[b, pl.ds(ci * HB, HB)], buf, sem).start()

        def wait(ci, buf, sem):
            pltpu.make_async_copy(
                x_hbm.at[b, pl.ds(ci * HB, HB)], buf, sem).wait()

        def accumulate(buf, carry):
            def pos(t, accs):
                accs = list(accs)
                hh = t // W
                ww = t - hh * W
                for cv in range(CV):
                    accs[cv] = accs[cv] + buf[hh, ww, pl.ds(cv * L, L)]
                return tuple(accs)
            return lax.fori_loop(0, HB * W, pos, carry, unroll=False)

        start(0, xb0, csem0)
        accs = (zero,) * CV

        def chunk_pair(g2, carry, ):
            ci = 2 * g2

            @pl.when(ci + 1 < NCH)
            def _():
                start(ci + 1, xb1, csem1)

            wait(ci, xb0, csem0)
            carry = accumulate(xb0, carry)

            @pl.when(ci + 2 < NCH)
            def _():
                start(ci + 2, xb0, csem0)

            wait(ci + 1, xb1, csem1)
            return accumulate(xb1, carry)

        accs = lax.fori_loop(0, NCH // 2, chunk_pair, accs, unroll=False)
        for cv in range(CV):
            pbuf[pl.ds(cv * L, L)] = accs[cv]

        inv = 1.0 / HW

        def h_acc(q, carry):
            hvs = list(carry)
            pv = pbuf[pl.ds(q * L, L)]
            for k in range(L):
                s = pv[k] * inv
                for j in range(RED // L):
                    hvs[j] = hvs[j] + s * w1v[q * L + k, pl.ds(j * L, L)]
            return tuple(hvs)

        hvs = lax.fori_loop(0, C // L, h_acc, (zero,) * (RED // L),
                            unroll=False)
        hjs = []
        for j in range(RED // L):
            hj = hvs[j] + b1v[pl.ds(j * L, L)]
            hj = hj / (1.0 + jnp.exp(-hj))
            hjs.append(hj)

        logits = b2v[...]
        for j in range(RED // L):
            for k in range(L):
                logits = logits + hjs[j][k] * w2v[j * L + k]

        ex = jnp.exp(logits)
        es = [ex[k] for k in range(E)]
        tot = es[0]
        for k in range(1, E):
            tot = tot + es[k]
        rtot = (jnp.ones((L,), jnp.float32) / jnp.full((L,), tot))[0]
        ws = [e * rtot for e in es]
        m1 = ws[0]
        for k in range(1, E):
            m1 = jnp.maximum(m1, ws[k])
        i1 = jnp.int32(E - 1)
        for k in reversed(range(E)):
            i1 = jnp.where(ws[k] == m1, jnp.int32(k), i1)
        wr = [jnp.where(i1 == k, -1.0, ws[k]) for k in range(E)]
        m2 = wr[0]
        for k in range(1, E):
            m2 = jnp.maximum(m2, wr[k])
        i2 = jnp.int32(E - 1)
        for k in reversed(range(E)):
            i2 = jnp.where(wr[k] == m2, jnp.int32(k), i2)
        keep = [(i1 == k) | (i2 == k) for k in range(E)]
        wm = [jnp.where(keep[k], ws[k], 0.0) for k in range(E)]
        den = wm[0]
        for k in range(1, E):
            den = den + wm[k]
        rden = (jnp.ones((L,), jnp.float32)
                / jnp.full((L,), den + 1e-8))[0]
        lanes = lax.iota(jnp.int32, L)
        wnv = zero
        for k in range(E):
            wnv = jnp.where(lanes == k, wm[k] * rden, wnv)
        ob[...] = wnv
        pltpu.sync_copy(ob, wn_hbm.at[wid])


def _bc_body(wn_tc_ref, wn_sc_ref, out_ref):
    wn_tc = wn_tc_ref[...]
    wn_sc = wn_sc_ref[...][:, :E]
    wn = jnp.concatenate([wn_tc, wn_sc], axis=0)
    wnt = wn.T
    out_ref[...] = jnp.broadcast_to(wnt[:, None, None, :], (E, H, W, B))


@jax.jit
def kernel(x, W1, b1, W2, b2):
    xt = jnp.transpose(x, (0, 2, 3, 1))
    w2p = jnp.pad(W2, ((0, 0), (0, L - E)))
    b2p = jnp.concatenate([b2, jnp.full((L - E,), -1e30, jnp.float32)])

    wn_tc = pl.pallas_call(
        _tc_body,
        grid=(BTC // BB,),
        in_specs=[
            pl.BlockSpec((BB, H, W, C), lambda i: (i, 0, 0, 0)),
            pl.BlockSpec((C, RED), lambda i: (0, 0)),
            pl.BlockSpec((1, RED), lambda i: (0, 0)),
            pl.BlockSpec((RED, E), lambda i: (0, 0)),
            pl.BlockSpec((1, E), lambda i: (0, 0)),
        ],
        out_specs=pl.BlockSpec((BB, E), lambda i: (i, 0)),
        out_shape=jax.ShapeDtypeStruct((BTC, E), jnp.float32),
    )(xt, W1, b1.reshape(1, RED), W2, b2.reshape(1, E))

    mesh = plsc.VectorSubcoreMesh(core_axis_name="c", subcore_axis_name="s")
    wn_sc = functools.partial(
        pl.kernel,
        out_type=jax.ShapeDtypeStruct((BSC, L), jnp.float32),
        mesh=mesh,
        compiler_params=pltpu.CompilerParams(
            needs_layout_passes=False, use_tc_tiling_on_sc=True),
        scratch_types=[
            pltpu.VMEM((HB, W, C), jnp.float32),
            pltpu.VMEM((HB, W, C), jnp.float32),
            pltpu.VMEM((C,), jnp.float32),
            pltpu.VMEM((C, RED), jnp.float32),
            pltpu.VMEM((RED,), jnp.float32),
            pltpu.VMEM((RED, L), jnp.float32),
            pltpu.VMEM((L,), jnp.float32),
            pltpu.VMEM((L,), jnp.float32),
            pltpu.SemaphoreType.DMA,
            pltpu.SemaphoreType.DMA,
        ],
    )(_sc_body)(xt, W1, b1, w2p, b2p)

    pout = pl.pallas_call(
        _bc_body,
        in_specs=[
            pl.BlockSpec((BTC, E), lambda: (0, 0)),
            pl.BlockSpec((BSC, L), lambda: (0, 0)),
        ],
        out_specs=pl.BlockSpec((E, H, W, B), lambda: (0, 0, 0, 0)),
        out_shape=jax.ShapeDtypeStruct((E, H, W, B), jnp.float32),
    )(wn_tc, wn_sc)
    return jnp.transpose(pout, (3, 0, 1, 2))


# speedup vs baseline: 2.2661x; 1.8241x over previous
import jax
import jax.numpy as jnp
from jax import lax
from jax.experimental import pallas as pl
from jax.experimental.pallas import tpu as pltpu

B, C, H, W = 64, 384, 32, 32
HW = H * W
E = 8
RED = 48
BB = 8


def _body(x_ref, w1_ref, b1_ref, w2_ref, b2_ref, out_ref, wn_ref):
    i = pl.program_id(0)
    xs = x_ref[...]
    n = H
    while n > 1:
        n //= 2
        xs = xs[:, :n] + xs[:, n:2 * n]
    ys = xs[:, 0]
    n = W
    while n > 1:
        n //= 2
        ys = ys[:, :n] + ys[:, n:2 * n]
    pooled = ys[:, 0] * (1.0 / HW)
    h = jnp.dot(pooled, w1_ref[...], preferred_element_type=jnp.float32)
    h = h + b1_ref[...]
    h = h * jax.nn.sigmoid(h)
    logits = jnp.dot(h, w2_ref[...], preferred_element_type=jnp.float32)
    logits = logits + b2_ref[...]
    w = jax.nn.softmax(logits, axis=1)
    idx = lax.broadcasted_iota(jnp.int32, (BB, E), 1)
    m1 = jnp.max(w, axis=1, keepdims=True)
    i1 = jnp.min(jnp.where(w == m1, idx, E), axis=1, keepdims=True)
    w_rest = jnp.where(idx == i1, -jnp.inf, w)
    m2 = jnp.max(w_rest, axis=1, keepdims=True)
    i2 = jnp.min(jnp.where(w_rest == m2, idx, E), axis=1, keepdims=True)
    mask = (idx == i1) | (idx == i2)
    wsel = jnp.where(mask, w, 0.0)
    wn = wsel / (jnp.sum(wsel, axis=1, keepdims=True) + 1e-8)
    wn_ref[pl.ds(i * BB, BB), :] = wn

    @pl.when(i == B // BB - 1)
    def _():
        wnt = wn_ref[...].T
        out_ref[...] = jnp.broadcast_to(wnt[:, None, None, :], (E, H, W, B))


@jax.jit
def kernel(x, W1, b1, W2, b2):
    xt = jnp.transpose(x, (0, 2, 3, 1))
    pout = pl.pallas_call(
        _body,
        grid=(B // BB,),
        in_specs=[
            pl.BlockSpec((BB, H, W, C), lambda i: (i, 0, 0, 0)),
            pl.BlockSpec((C, RED), lambda i: (0, 0)),
            pl.BlockSpec((1, RED), lambda i: (0, 0)),
            pl.BlockSpec((RED, E), lambda i: (0, 0)),
            pl.BlockSpec((1, E), lambda i: (0, 0)),
        ],
        out_specs=pl.BlockSpec((E, H, W, B), lambda i: (0, 0, 0, 0)),
        out_shape=jax.ShapeDtypeStruct((E, H, W, B), jnp.float32),
        scratch_shapes=[pltpu.VMEM((B, E), jnp.float32)],
    )(xt, W1, b1.reshape(1, RED), W2, b2.reshape(1, E))
    return jnp.transpose(pout, (3, 0, 1, 2))
